# baseline (speedup 1.0000x reference)
import functools

import jax
import jax.numpy as jnp
from jax import lax
from jax.experimental import pallas as pl
from jax.experimental.pallas import tpu as pltpu
from jax.experimental.pallas import tpu_sc as plsc

B = 16384
D = 64

_info = plsc.get_sparse_core_info()
_NC, _NS = _info.num_cores, _info.num_subcores
_NW = _NC * _NS
_BPW = B // _NW
_CHUNK = 128
_NCHUNK = _BPW // _CHUNK

_sc_mesh = plsc.VectorSubcoreMesh(core_axis_name="c", subcore_axis_name="s")


@functools.partial(
    pl.kernel,
    mesh=_sc_mesh,
    out_type=[
        jax.ShapeDtypeStruct((B, 2 * D), jnp.float32),
        jax.ShapeDtypeStruct((B, 2 * D), jnp.float32),
    ],
    scratch_types=[
        pltpu.VMEM((_NCHUNK, _CHUNK), jnp.int32),
        pltpu.VMEM((_BPW, 2 * D), jnp.float32),
        pltpu.SemaphoreType.DMA,
    ],
)
def _sc_gather(uid_hbm, iid_hbm, utab_hbm, itab_hbm, u_out, i_out,
               idx_v, rows_v, sem):
    wid = lax.axis_index("s") * _NC + lax.axis_index("c")
    base = wid * _BPW
    for tab_hbm, ids_hbm, out_hbm in ((utab_hbm, uid_hbm, u_out),
                                      (itab_hbm, iid_hbm, i_out)):
        for j in range(_NCHUNK):
            pltpu.sync_copy(ids_hbm.at[pl.ds(base + j * _CHUNK, _CHUNK)],
                            idx_v.at[j])
        copies = [
            pltpu.async_copy(tab_hbm.at[idx_v.at[j]],
                             rows_v.at[pl.ds(j * _CHUNK, _CHUNK)], sem)
            for j in range(_NCHUNK)
        ]
        for c in copies:
            c.wait()
        pltpu.sync_copy(rows_v, out_hbm.at[pl.ds(base, _BPW)])


def _mlp_body(u2_ref, i2_ref, pu_ref, pi_ref, w1u_ref, w1i_ref, b1_ref,
              w2_ref, b2_ref, w3_ref, b3_ref, w4_ref, b4_ref, o_ref):
    pu = pu_ref[...]
    pi = pi_ref[...]
    u = u2_ref[:, :D] * (1.0 - pu) + u2_ref[:, D:] * pu
    i = i2_ref[:, :D] * (1.0 - pi) + i2_ref[:, D:] * pi
    h = jnp.dot(u, w1u_ref[...], preferred_element_type=jnp.float32)
    h = h + jnp.dot(i, w1i_ref[...], preferred_element_type=jnp.float32)
    h = jnp.maximum(h + b1_ref[...], 0.0)
    h = jnp.maximum(
        jnp.dot(h, w2_ref[...], preferred_element_type=jnp.float32)
        + b2_ref[...], 0.0)
    h = jnp.maximum(
        jnp.dot(h, w3_ref[...], preferred_element_type=jnp.float32)
        + b3_ref[...], 0.0)
    z = (jnp.dot(h, w4_ref[...], preferred_element_type=jnp.float32)
         + b4_ref[...])
    o_ref[...] = jax.nn.sigmoid(z)


_BBLK = 2048


def _mlp(u2, i2, pu, pi, w1u, w1i, b1, w2, b2, w3, b3, w4, b4):
    grid = (B // _BBLK,)
    full = lambda gi: (0, 0)
    row = lambda gi: (gi, 0)
    return pl.pallas_call(
        _mlp_body,
        grid=grid,
        in_specs=[
            pl.BlockSpec((_BBLK, 2 * D), row),
            pl.BlockSpec((_BBLK, 2 * D), row),
            pl.BlockSpec((_BBLK, 1), row),
            pl.BlockSpec((_BBLK, 1), row),
            pl.BlockSpec((D, 128), full),
            pl.BlockSpec((D, 128), full),
            pl.BlockSpec((1, 128), full),
            pl.BlockSpec((128, 64), full),
            pl.BlockSpec((1, 64), full),
            pl.BlockSpec((64, 32), full),
            pl.BlockSpec((1, 32), full),
            pl.BlockSpec((32, 1), full),
            pl.BlockSpec((1, 1), full),
        ],
        out_specs=pl.BlockSpec((_BBLK, 1), row),
        out_shape=jax.ShapeDtypeStruct((B, 1), jnp.float32),
    )(u2, i2, pu, pi, w1u, w1i, b1, w2, b2, w3, b3, w4, b4)


def kernel(user_ids, item_ids, user_table, item_table,
           W1, b1, W2, b2, W3, b3, W4, b4):
    uids = user_ids.astype(jnp.int32)
    iids = item_ids.astype(jnp.int32)
    utab2 = user_table.reshape(-1, 2 * D)
    itab2 = item_table.reshape(-1, 2 * D)
    u2, i2 = _sc_gather(uids // 2, iids // 2, utab2, itab2)
    pu = (uids % 2).astype(jnp.float32)[:, None]
    pi = (iids % 2).astype(jnp.float32)[:, None]
    w1t = W1.T
    out = _mlp(u2, i2, pu, pi, w1t[:D], w1t[D:], b1[None, :], W2.T,
               b2[None, :], W3.T, b3[None, :], W4.T, b4[None, :])
    return out[:, 0]


# speedup vs baseline: 1.3853x; 1.3853x over previous
import functools

import jax
import jax.numpy as jnp
from jax import lax
from jax.experimental import pallas as pl
from jax.experimental.pallas import tpu as pltpu
from jax.experimental.pallas import tpu_sc as plsc

B = 16384
D = 64
NROWS = 1000000

_info = plsc.get_sparse_core_info()
_NC, _NS = _info.num_cores, _info.num_subcores
_NW = _NC * _NS
_BPW = B // _NW
_CHUNK = 128
_NCHUNK = _BPW // _CHUNK

_sc_mesh = plsc.VectorSubcoreMesh(core_axis_name="c", subcore_axis_name="s")


_TBLK = 2048
_HBLK = _TBLK // 2
_TGRID = pl.cdiv(NROWS, _TBLK)
NPAIR = _TGRID * _HBLK


def _tr_body(xT_ref, o_ref):
    xs = jnp.concatenate([xT_ref[:, :_HBLK], xT_ref[:, _HBLK:]], axis=0)
    o_ref[...] = xs.T


def _to_pairs(tabT):
    return pl.pallas_call(
        _tr_body,
        grid=(_TGRID,),
        in_specs=[pl.BlockSpec((D, _TBLK), lambda g: (0, g))],
        out_specs=pl.BlockSpec((_HBLK, 2 * D), lambda g: (g, 0)),
        out_shape=jax.ShapeDtypeStruct((NPAIR, 2 * D), jnp.float32),
    )(tabT)


@functools.partial(
    pl.kernel,
    mesh=_sc_mesh,
    out_type=[
        jax.ShapeDtypeStruct((B, 2 * D), jnp.float32),
        jax.ShapeDtypeStruct((B, 2 * D), jnp.float32),
    ],
    scratch_types=[
        pltpu.VMEM((_NCHUNK, _CHUNK), jnp.int32),
        pltpu.VMEM((_BPW, 2 * D), jnp.float32),
        pltpu.SemaphoreType.DMA,
    ],
)
def _sc_gather(uid_hbm, iid_hbm, utab_hbm, itab_hbm, u_out, i_out,
               idx_v, rows_v, sem):
    wid = lax.axis_index("s") * _NC + lax.axis_index("c")
    base = wid * _BPW
    for tab_hbm, ids_hbm, out_hbm in ((utab_hbm, uid_hbm, u_out),
                                      (itab_hbm, iid_hbm, i_out)):
        for j in range(_NCHUNK):
            pltpu.sync_copy(ids_hbm.at[pl.ds(base + j * _CHUNK, _CHUNK)],
                            idx_v.at[j])
        copies = [
            pltpu.async_copy(tab_hbm.at[idx_v.at[j]],
                             rows_v.at[pl.ds(j * _CHUNK, _CHUNK)], sem)
            for j in range(_NCHUNK)
        ]
        for c in copies:
            c.wait()
        pltpu.sync_copy(rows_v, out_hbm.at[pl.ds(base, _BPW)])


def _mlp_body(u2_ref, i2_ref, pu_ref, pi_ref, w1u_ref, w1i_ref, b1_ref,
              w2_ref, b2_ref, w3_ref, b3_ref, w4_ref, b4_ref, o_ref):
    pu = pu_ref[...]
    pi = pi_ref[...]
    u = u2_ref[:, :D] * (1.0 - pu) + u2_ref[:, D:] * pu
    i = i2_ref[:, :D] * (1.0 - pi) + i2_ref[:, D:] * pi
    h = jnp.dot(u, w1u_ref[...], preferred_element_type=jnp.float32)
    h = h + jnp.dot(i, w1i_ref[...], preferred_element_type=jnp.float32)
    h = jnp.maximum(h + b1_ref[...], 0.0)
    h = jnp.maximum(
        jnp.dot(h, w2_ref[...], preferred_element_type=jnp.float32)
        + b2_ref[...], 0.0)
    h = jnp.maximum(
        jnp.dot(h, w3_ref[...], preferred_element_type=jnp.float32)
        + b3_ref[...], 0.0)
    z = (jnp.dot(h, w4_ref[...], preferred_element_type=jnp.float32)
         + b4_ref[...])
    o_ref[...] = jax.nn.sigmoid(z)


_BBLK = 2048


def _mlp(u2, i2, pu, pi, w1u, w1i, b1, w2, b2, w3, b3, w4, b4):
    grid = (B // _BBLK,)
    full = lambda gi: (0, 0)
    row = lambda gi: (gi, 0)
    return pl.pallas_call(
        _mlp_body,
        grid=grid,
        in_specs=[
            pl.BlockSpec((_BBLK, 2 * D), row),
            pl.BlockSpec((_BBLK, 2 * D), row),
            pl.BlockSpec((_BBLK, 1), row),
            pl.BlockSpec((_BBLK, 1), row),
            pl.BlockSpec((D, 128), full),
            pl.BlockSpec((D, 128), full),
            pl.BlockSpec((1, 128), full),
            pl.BlockSpec((128, 64), full),
            pl.BlockSpec((1, 64), full),
            pl.BlockSpec((64, 32), full),
            pl.BlockSpec((1, 32), full),
            pl.BlockSpec((32, 1), full),
            pl.BlockSpec((1, 1), full),
        ],
        out_specs=pl.BlockSpec((_BBLK, 1), row),
        out_shape=jax.ShapeDtypeStruct((B, 1), jnp.float32),
    )(u2, i2, pu, pi, w1u, w1i, b1, w2, b2, w3, b3, w4, b4)


def kernel(user_ids, item_ids, user_table, item_table,
           W1, b1, W2, b2, W3, b3, W4, b4):
    uids = user_ids.astype(jnp.int32)
    iids = item_ids.astype(jnp.int32)
    utab2 = _to_pairs(user_table.T)
    itab2 = _to_pairs(item_table.T)
    upair = (uids // _TBLK) * _HBLK + uids % _HBLK
    ipair = (iids // _TBLK) * _HBLK + iids % _HBLK
    u2, i2 = _sc_gather(upair, ipair, utab2, itab2)
    pu = ((uids % _TBLK) // _HBLK).astype(jnp.float32)[:, None]
    pi = ((iids % _TBLK) // _HBLK).astype(jnp.float32)[:, None]
    w1t = W1.T
    out = _mlp(u2, i2, pu, pi, w1t[:D], w1t[D:], b1[None, :], W2.T,
               b2[None, :], W3.T, b3[None, :], W4.T, b4[None, :])
    return out[:, 0]


# speedup vs baseline: 2.5445x; 1.8368x over previous
import functools

import jax
import jax.numpy as jnp
from jax import lax
from jax.experimental import pallas as pl
from jax.experimental.pallas import tpu as pltpu
from jax.experimental.pallas import tpu_sc as plsc

B = 16384
D = 64
NROWS = 1000000

_info = plsc.get_sparse_core_info()
_NC, _NS = _info.num_cores, _info.num_subcores
_NW = _NC * _NS
_BPW = B // _NW
_CHUNK = 128
_NCHUNK = _BPW // _CHUNK

_sc_mesh = plsc.VectorSubcoreMesh(core_axis_name="c", subcore_axis_name="s")


_TBLK = 8192
_HBLK = _TBLK // 2
_TGRID = pl.cdiv(NROWS, _TBLK)
NPAIR = _TGRID * _HBLK


def _tr_body(xT_ref, o_ref):
    xs = jnp.concatenate([xT_ref[:, :_HBLK], xT_ref[:, _HBLK:]], axis=0)
    o_ref[...] = xs.T


def _to_pairs(tabT):
    return pl.pallas_call(
        _tr_body,
        grid=(_TGRID,),
        in_specs=[pl.BlockSpec((D, _TBLK), lambda g: (0, g))],
        out_specs=pl.BlockSpec((_HBLK, 2 * D), lambda g: (g, 0)),
        out_shape=jax.ShapeDtypeStruct((NPAIR, 2 * D), jnp.float32),
    )(tabT)


@functools.partial(
    pl.kernel,
    mesh=_sc_mesh,
    out_type=[
        jax.ShapeDtypeStruct((B, 2 * D), jnp.float32),
        jax.ShapeDtypeStruct((B, 2 * D), jnp.float32),
    ],
    scratch_types=[
        pltpu.VMEM((_NCHUNK, _CHUNK), jnp.int32),
        pltpu.VMEM((_BPW, 2 * D), jnp.float32),
        pltpu.SemaphoreType.DMA,
    ],
)
def _sc_gather(uid_hbm, iid_hbm, utab_hbm, itab_hbm, u_out, i_out,
               idx_v, rows_v, sem):
    wid = lax.axis_index("s") * _NC + lax.axis_index("c")
    base = wid * _BPW
    for tab_hbm, ids_hbm, out_hbm in ((utab_hbm, uid_hbm, u_out),
                                      (itab_hbm, iid_hbm, i_out)):
        for j in range(_NCHUNK):
            pltpu.sync_copy(ids_hbm.at[pl.ds(base + j * _CHUNK, _CHUNK)],
                            idx_v.at[j])
        copies = [
            pltpu.async_copy(tab_hbm.at[idx_v.at[j]],
                             rows_v.at[pl.ds(j * _CHUNK, _CHUNK)], sem)
            for j in range(_NCHUNK)
        ]
        for c in copies:
            c.wait()
        pltpu.sync_copy(rows_v, out_hbm.at[pl.ds(base, _BPW)])


def _mlp_body(u2_ref, i2_ref, pu_ref, pi_ref, w1u_ref, w1i_ref, b1_ref,
              w2_ref, b2_ref, w3_ref, b3_ref, w4_ref, b4_ref, o_ref):
    pu = pu_ref[...]
    pi = pi_ref[...]
    u = u2_ref[:, :D] * (1.0 - pu) + u2_ref[:, D:] * pu
    i = i2_ref[:, :D] * (1.0 - pi) + i2_ref[:, D:] * pi
    h = jnp.dot(u, w1u_ref[...], preferred_element_type=jnp.float32)
    h = h + jnp.dot(i, w1i_ref[...], preferred_element_type=jnp.float32)
    h = jnp.maximum(h + b1_ref[...], 0.0)
    h = jnp.maximum(
        jnp.dot(h, w2_ref[...], preferred_element_type=jnp.float32)
        + b2_ref[...], 0.0)
    h = jnp.maximum(
        jnp.dot(h, w3_ref[...], preferred_element_type=jnp.float32)
        + b3_ref[...], 0.0)
    z = (jnp.dot(h, w4_ref[...], preferred_element_type=jnp.float32)
         + b4_ref[...])
    o_ref[...] = jax.nn.sigmoid(z)


_BBLK = 2048


def _mlp(u2, i2, pu, pi, w1u, w1i, b1, w2, b2, w3, b3, w4, b4):
    grid = (B // _BBLK,)
    full = lambda gi: (0, 0)
    row = lambda gi: (gi, 0)
    return pl.pallas_call(
        _mlp_body,
        grid=grid,
        in_specs=[
            pl.BlockSpec((_BBLK, 2 * D), row),
            pl.BlockSpec((_BBLK, 2 * D), row),
            pl.BlockSpec((_BBLK, 1), row),
            pl.BlockSpec((_BBLK, 1), row),
            pl.BlockSpec((D, 128), full),
            pl.BlockSpec((D, 128), full),
            pl.BlockSpec((1, 128), full),
            pl.BlockSpec((128, 64), full),
            pl.BlockSpec((1, 64), full),
            pl.BlockSpec((64, 32), full),
            pl.BlockSpec((1, 32), full),
            pl.BlockSpec((32, 1), full),
            pl.BlockSpec((1, 1), full),
        ],
        out_specs=pl.BlockSpec((_BBLK, 1), row),
        out_shape=jax.ShapeDtypeStruct((B, 1), jnp.float32),
    )(u2, i2, pu, pi, w1u, w1i, b1, w2, b2, w3, b3, w4, b4)


def kernel(user_ids, item_ids, user_table, item_table,
           W1, b1, W2, b2, W3, b3, W4, b4):
    uids = user_ids.astype(jnp.int32)
    iids = item_ids.astype(jnp.int32)
    utab2 = _to_pairs(user_table.T)
    itab2 = _to_pairs(item_table.T)
    upair = (uids // _TBLK) * _HBLK + uids % _HBLK
    ipair = (iids // _TBLK) * _HBLK + iids % _HBLK
    u2, i2 = _sc_gather(upair, ipair, utab2, itab2)
    pu = ((uids % _TBLK) // _HBLK).astype(jnp.float32)[:, None]
    pi = ((iids % _TBLK) // _HBLK).astype(jnp.float32)[:, None]
    w1t = W1.T
    out = _mlp(u2, i2, pu, pi, w1t[:D], w1t[D:], b1[None, :], W2.T,
               b2[None, :], W3.T, b3[None, :], W4.T, b4[None, :])
    return out[:, 0]


# speedup vs baseline: 2.8883x; 1.1351x over previous
import functools

import jax
import jax.numpy as jnp
from jax import lax
from jax.experimental import pallas as pl
from jax.experimental.pallas import tpu as pltpu
from jax.experimental.pallas import tpu_sc as plsc

B = 16384
D = 64
NROWS = 1000000

_info = plsc.get_sparse_core_info()
_NC, _NS = _info.num_cores, _info.num_subcores
_NW = _NC * _NS
_BPW = B // _NW
_CHUNK = 128
_NCHUNK = _BPW // _CHUNK

_sc_mesh = plsc.VectorSubcoreMesh(core_axis_name="c", subcore_axis_name="s")


_TBLK = 16384
_HBLK = _TBLK // 2
_TGRID = pl.cdiv(NROWS, _TBLK)
NPAIR = _TGRID * _HBLK


def _tr_body(xT_ref, o_ref):
    xs = jnp.concatenate([xT_ref[:, :_HBLK], xT_ref[:, _HBLK:]], axis=0)
    o_ref[...] = xs.T


def _to_pairs(tabT):
    return pl.pallas_call(
        _tr_body,
        grid=(_TGRID,),
        in_specs=[pl.BlockSpec((D, _TBLK), lambda g: (0, g))],
        out_specs=pl.BlockSpec((_HBLK, 2 * D), lambda g: (g, 0)),
        out_shape=jax.ShapeDtypeStruct((NPAIR, 2 * D), jnp.float32),
    )(tabT)


@functools.partial(
    pl.kernel,
    mesh=_sc_mesh,
    out_type=[
        jax.ShapeDtypeStruct((B, 2 * D), jnp.float32),
        jax.ShapeDtypeStruct((B, 2 * D), jnp.float32),
    ],
    scratch_types=[
        pltpu.VMEM((_NCHUNK, _CHUNK), jnp.int32),
        pltpu.VMEM((_BPW, 2 * D), jnp.float32),
        pltpu.SemaphoreType.DMA,
    ],
)
def _sc_gather(uid_hbm, iid_hbm, utab_hbm, itab_hbm, u_out, i_out,
               idx_v, rows_v, sem):
    wid = lax.axis_index("s") * _NC + lax.axis_index("c")
    base = wid * _BPW
    for tab_hbm, ids_hbm, out_hbm in ((utab_hbm, uid_hbm, u_out),
                                      (itab_hbm, iid_hbm, i_out)):
        for j in range(_NCHUNK):
            pltpu.sync_copy(ids_hbm.at[pl.ds(base + j * _CHUNK, _CHUNK)],
                            idx_v.at[j])
        copies = [
            pltpu.async_copy(tab_hbm.at[idx_v.at[j]],
                             rows_v.at[pl.ds(j * _CHUNK, _CHUNK)], sem)
            for j in range(_NCHUNK)
        ]
        for c in copies:
            c.wait()
        pltpu.sync_copy(rows_v, out_hbm.at[pl.ds(base, _BPW)])


def _mlp_body(u2_ref, i2_ref, pu_ref, pi_ref, w1u_ref, w1i_ref, b1_ref,
              w2_ref, b2_ref, w3_ref, b3_ref, w4_ref, b4_ref, o_ref):
    pu = pu_ref[...]
    pi = pi_ref[...]
    u = u2_ref[:, :D] * (1.0 - pu) + u2_ref[:, D:] * pu
    i = i2_ref[:, :D] * (1.0 - pi) + i2_ref[:, D:] * pi
    h = jnp.dot(u, w1u_ref[...], preferred_element_type=jnp.float32)
    h = h + jnp.dot(i, w1i_ref[...], preferred_element_type=jnp.float32)
    h = jnp.maximum(h + b1_ref[...], 0.0)
    h = jnp.maximum(
        jnp.dot(h, w2_ref[...], preferred_element_type=jnp.float32)
        + b2_ref[...], 0.0)
    h = jnp.maximum(
        jnp.dot(h, w3_ref[...], preferred_element_type=jnp.float32)
        + b3_ref[...], 0.0)
    z = (jnp.dot(h, w4_ref[...], preferred_element_type=jnp.float32)
         + b4_ref[...])
    o_ref[...] = jax.nn.sigmoid(z)


_BBLK = 2048


def _mlp(u2, i2, pu, pi, w1u, w1i, b1, w2, b2, w3, b3, w4, b4):
    grid = (B // _BBLK,)
    full = lambda gi: (0, 0)
    row = lambda gi: (gi, 0)
    return pl.pallas_call(
        _mlp_body,
        grid=grid,
        in_specs=[
            pl.BlockSpec((_BBLK, 2 * D), row),
            pl.BlockSpec((_BBLK, 2 * D), row),
            pl.BlockSpec((_BBLK, 1), row),
            pl.BlockSpec((_BBLK, 1), row),
            pl.BlockSpec((D, 128), full),
            pl.BlockSpec((D, 128), full),
            pl.BlockSpec((1, 128), full),
            pl.BlockSpec((128, 64), full),
            pl.BlockSpec((1, 64), full),
            pl.BlockSpec((64, 32), full),
            pl.BlockSpec((1, 32), full),
            pl.BlockSpec((32, 1), full),
            pl.BlockSpec((1, 1), full),
        ],
        out_specs=pl.BlockSpec((_BBLK, 1), row),
        out_shape=jax.ShapeDtypeStruct((B, 1), jnp.float32),
    )(u2, i2, pu, pi, w1u, w1i, b1, w2, b2, w3, b3, w4, b4)


def kernel(user_ids, item_ids, user_table, item_table,
           W1, b1, W2, b2, W3, b3, W4, b4):
    uids = user_ids.astype(jnp.int32)
    iids = item_ids.astype(jnp.int32)
    utab2 = _to_pairs(user_table.T)
    itab2 = _to_pairs(item_table.T)
    upair = (uids // _TBLK) * _HBLK + uids % _HBLK
    ipair = (iids // _TBLK) * _HBLK + iids % _HBLK
    u2, i2 = _sc_gather(upair, ipair, utab2, itab2)
    pu = ((uids % _TBLK) // _HBLK).astype(jnp.float32)[:, None]
    pi = ((iids % _TBLK) // _HBLK).astype(jnp.float32)[:, None]
    w1t = W1.T
    out = _mlp(u2, i2, pu, pi, w1t[:D], w1t[D:], b1[None, :], W2.T,
               b2[None, :], W3.T, b3[None, :], W4.T, b4[None, :])
    return out[:, 0]


# speedup vs baseline: 2.9597x; 1.0247x over previous
import functools

import jax
import jax.numpy as jnp
from jax import lax
from jax.experimental import pallas as pl
from jax.experimental.pallas import tpu as pltpu
from jax.experimental.pallas import tpu_sc as plsc

B = 16384
D = 64
NROWS = 1000000

_info = plsc.get_sparse_core_info()
_NC, _NS = _info.num_cores, _info.num_subcores
_NW = _NC * _NS
_BPW = B // _NW
_CHUNK = 128
_NCHUNK = _BPW // _CHUNK

_sc_mesh = plsc.VectorSubcoreMesh(core_axis_name="c", subcore_axis_name="s")


_TBLK = 32768
_HBLK = _TBLK // 2
_TGRID = pl.cdiv(NROWS, _TBLK)
NPAIR = _TGRID * _HBLK


def _tr_body(xT_ref, o_ref):
    xs = jnp.concatenate([xT_ref[:, :_HBLK], xT_ref[:, _HBLK:]], axis=0)
    o_ref[...] = xs.T


def _to_pairs(tabT):
    return pl.pallas_call(
        _tr_body,
        grid=(_TGRID,),
        in_specs=[pl.BlockSpec((D, _TBLK), lambda g: (0, g))],
        out_specs=pl.BlockSpec((_HBLK, 2 * D), lambda g: (g, 0)),
        out_shape=jax.ShapeDtypeStruct((NPAIR, 2 * D), jnp.float32),
    )(tabT)


@functools.partial(
    pl.kernel,
    mesh=_sc_mesh,
    out_type=[
        jax.ShapeDtypeStruct((B, 2 * D), jnp.float32),
        jax.ShapeDtypeStruct((B, 2 * D), jnp.float32),
    ],
    scratch_types=[
        pltpu.VMEM((_NCHUNK, _CHUNK), jnp.int32),
        pltpu.VMEM((_BPW, 2 * D), jnp.float32),
        pltpu.SemaphoreType.DMA,
    ],
)
def _sc_gather(uid_hbm, iid_hbm, utab_hbm, itab_hbm, u_out, i_out,
               idx_v, rows_v, sem):
    wid = lax.axis_index("s") * _NC + lax.axis_index("c")
    base = wid * _BPW
    for tab_hbm, ids_hbm, out_hbm in ((utab_hbm, uid_hbm, u_out),
                                      (itab_hbm, iid_hbm, i_out)):
        for j in range(_NCHUNK):
            pltpu.sync_copy(ids_hbm.at[pl.ds(base + j * _CHUNK, _CHUNK)],
                            idx_v.at[j])
        copies = [
            pltpu.async_copy(tab_hbm.at[idx_v.at[j]],
                             rows_v.at[pl.ds(j * _CHUNK, _CHUNK)], sem)
            for j in range(_NCHUNK)
        ]
        for c in copies:
            c.wait()
        pltpu.sync_copy(rows_v, out_hbm.at[pl.ds(base, _BPW)])


def _mlp_body(u2_ref, i2_ref, pu_ref, pi_ref, w1u_ref, w1i_ref, b1_ref,
              w2_ref, b2_ref, w3_ref, b3_ref, w4_ref, b4_ref, o_ref):
    pu = pu_ref[...]
    pi = pi_ref[...]
    u = u2_ref[:, :D] * (1.0 - pu) + u2_ref[:, D:] * pu
    i = i2_ref[:, :D] * (1.0 - pi) + i2_ref[:, D:] * pi
    h = jnp.dot(u, w1u_ref[...], preferred_element_type=jnp.float32)
    h = h + jnp.dot(i, w1i_ref[...], preferred_element_type=jnp.float32)
    h = jnp.maximum(h + b1_ref[...], 0.0)
    h = jnp.maximum(
        jnp.dot(h, w2_ref[...], preferred_element_type=jnp.float32)
        + b2_ref[...], 0.0)
    h = jnp.maximum(
        jnp.dot(h, w3_ref[...], preferred_element_type=jnp.float32)
        + b3_ref[...], 0.0)
    z = (jnp.dot(h, w4_ref[...], preferred_element_type=jnp.float32)
         + b4_ref[...])
    o_ref[...] = jax.nn.sigmoid(z)


_BBLK = 2048


def _mlp(u2, i2, pu, pi, w1u, w1i, b1, w2, b2, w3, b3, w4, b4):
    grid = (B // _BBLK,)
    full = lambda gi: (0, 0)
    row = lambda gi: (gi, 0)
    return pl.pallas_call(
        _mlp_body,
        grid=grid,
        in_specs=[
            pl.BlockSpec((_BBLK, 2 * D), row),
            pl.BlockSpec((_BBLK, 2 * D), row),
            pl.BlockSpec((_BBLK, 1), row),
            pl.BlockSpec((_BBLK, 1), row),
            pl.BlockSpec((D, 128), full),
            pl.BlockSpec((D, 128), full),
            pl.BlockSpec((1, 128), full),
            pl.BlockSpec((128, 64), full),
            pl.BlockSpec((1, 64), full),
            pl.BlockSpec((64, 32), full),
            pl.BlockSpec((1, 32), full),
            pl.BlockSpec((32, 1), full),
            pl.BlockSpec((1, 1), full),
        ],
        out_specs=pl.BlockSpec((_BBLK, 1), row),
        out_shape=jax.ShapeDtypeStruct((B, 1), jnp.float32),
    )(u2, i2, pu, pi, w1u, w1i, b1, w2, b2, w3, b3, w4, b4)


def kernel(user_ids, item_ids, user_table, item_table,
           W1, b1, W2, b2, W3, b3, W4, b4):
    uids = user_ids.astype(jnp.int32)
    iids = item_ids.astype(jnp.int32)
    utab2 = _to_pairs(user_table.T)
    itab2 = _to_pairs(item_table.T)
    upair = (uids // _TBLK) * _HBLK + uids % _HBLK
    ipair = (iids // _TBLK) * _HBLK + iids % _HBLK
    u2, i2 = _sc_gather(upair, ipair, utab2, itab2)
    pu = ((uids % _TBLK) // _HBLK).astype(jnp.float32)[:, None]
    pi = ((iids % _TBLK) // _HBLK).astype(jnp.float32)[:, None]
    w1t = W1.T
    out = _mlp(u2, i2, pu, pi, w1t[:D], w1t[D:], b1[None, :], W2.T,
               b2[None, :], W3.T, b3[None, :], W4.T, b4[None, :])
    return out[:, 0]


# speedup vs baseline: 3.3689x; 1.1383x over previous
import functools

import jax
import jax.numpy as jnp
from jax import lax
from jax.experimental import pallas as pl
from jax.experimental.pallas import tpu as pltpu
from jax.experimental.pallas import tpu_sc as plsc

B = 16384
D = 64
NROWS = 1000000

_info = plsc.get_sparse_core_info()
_NC, _NS = _info.num_cores, _info.num_subcores
_NW = _NC * _NS
_BPW = B // _NW
_CHUNK = 128
_NCHUNK = _BPW // _CHUNK

_sc_mesh = plsc.VectorSubcoreMesh(core_axis_name="c", subcore_axis_name="s")


_TBLK = 32768
_QBLK = _TBLK // 4
_TGRID = pl.cdiv(NROWS, _TBLK)
NQUAD = _TGRID * _QBLK

_HI = 0xFFFF0000


def _bf16_hi_bits(x):
    y = x.astype(jnp.bfloat16).astype(jnp.float32)
    return jax.lax.bitcast_convert_type(y, jnp.uint32)


def _tr_body(xT_ref, o_ref):
    x = xT_ref[...]
    xs = jnp.concatenate(
        [x[:, i * _QBLK:(i + 1) * _QBLK] for i in range(4)], axis=0)
    t = xs.T
    a = _bf16_hi_bits(t[:, :D])
    b = _bf16_hi_bits(t[:, D:2 * D])
    c = _bf16_hi_bits(t[:, 2 * D:3 * D])
    d = _bf16_hi_bits(t[:, 3 * D:])
    wab = (a >> jnp.uint32(16)) | (b & jnp.uint32(_HI))
    wcd = (c >> jnp.uint32(16)) | (d & jnp.uint32(_HI))
    packed = jnp.concatenate([wab, wcd], axis=1)
    o_ref[...] = jax.lax.bitcast_convert_type(packed, jnp.float32)


def _to_quads(tabT):
    return pl.pallas_call(
        _tr_body,
        grid=(_TGRID,),
        in_specs=[pl.BlockSpec((D, _TBLK), lambda g: (0, g))],
        out_specs=pl.BlockSpec((_QBLK, 2 * D), lambda g: (g, 0)),
        out_shape=jax.ShapeDtypeStruct((NQUAD, 2 * D), jnp.float32),
    )(tabT)


@functools.partial(
    pl.kernel,
    mesh=_sc_mesh,
    out_type=[
        jax.ShapeDtypeStruct((B, 2 * D), jnp.float32),
        jax.ShapeDtypeStruct((B, 2 * D), jnp.float32),
    ],
    scratch_types=[
        pltpu.VMEM((_NCHUNK, _CHUNK), jnp.int32),
        pltpu.VMEM((_BPW, 2 * D), jnp.float32),
        pltpu.SemaphoreType.DMA,
    ],
)
def _sc_gather(uid_hbm, iid_hbm, utab_hbm, itab_hbm, u_out, i_out,
               idx_v, rows_v, sem):
    wid = lax.axis_index("s") * _NC + lax.axis_index("c")
    base = wid * _BPW
    for tab_hbm, ids_hbm, out_hbm in ((utab_hbm, uid_hbm, u_out),
                                      (itab_hbm, iid_hbm, i_out)):
        for j in range(_NCHUNK):
            pltpu.sync_copy(ids_hbm.at[pl.ds(base + j * _CHUNK, _CHUNK)],
                            idx_v.at[j])
        copies = [
            pltpu.async_copy(tab_hbm.at[idx_v.at[j]],
                             rows_v.at[pl.ds(j * _CHUNK, _CHUNK)], sem)
            for j in range(_NCHUNK)
        ]
        for c in copies:
            c.wait()
        pltpu.sync_copy(rows_v, out_hbm.at[pl.ds(base, _BPW)])


def _unpack(v2, wsel, lohi):
    v = jax.lax.bitcast_convert_type(v2, jnp.uint32)
    half = jnp.where(wsel, v[:, D:], v[:, :D])
    bits = jnp.where(lohi, half & jnp.uint32(_HI), half << jnp.uint32(16))
    return jax.lax.bitcast_convert_type(bits, jnp.float32)


def _mlp_body(u2_ref, i2_ref, su_ref, si_ref, w1u_ref, w1i_ref, b1_ref,
              w2_ref, b2_ref, w3_ref, b3_ref, w4_ref, b4_ref, o_ref):
    su = su_ref[...]
    si = si_ref[...]
    u = _unpack(u2_ref[...], su >= 2, (su % 2) == 1)
    i = _unpack(i2_ref[...], si >= 2, (si % 2) == 1)
    h = jnp.dot(u, w1u_ref[...], preferred_element_type=jnp.float32)
    h = h + jnp.dot(i, w1i_ref[...], preferred_element_type=jnp.float32)
    h = jnp.maximum(h + b1_ref[...], 0.0)
    h = jnp.maximum(
        jnp.dot(h, w2_ref[...], preferred_element_type=jnp.float32)
        + b2_ref[...], 0.0)
    h = jnp.maximum(
        jnp.dot(h, w3_ref[...], preferred_element_type=jnp.float32)
        + b3_ref[...], 0.0)
    z = (jnp.dot(h, w4_ref[...], preferred_element_type=jnp.float32)
         + b4_ref[...])
    o_ref[...] = jax.nn.sigmoid(z)


_BBLK = 2048


def _mlp(u2, i2, su, si, w1u, w1i, b1, w2, b2, w3, b3, w4, b4):
    grid = (B // _BBLK,)
    full = lambda gi: (0, 0)
    row = lambda gi: (gi, 0)
    return pl.pallas_call(
        _mlp_body,
        grid=grid,
        in_specs=[
            pl.BlockSpec((_BBLK, 2 * D), row),
            pl.BlockSpec((_BBLK, 2 * D), row),
            pl.BlockSpec((_BBLK, 1), row),
            pl.BlockSpec((_BBLK, 1), row),
            pl.BlockSpec((D, 128), full),
            pl.BlockSpec((D, 128), full),
            pl.BlockSpec((1, 128), full),
            pl.BlockSpec((128, 64), full),
            pl.BlockSpec((1, 64), full),
            pl.BlockSpec((64, 32), full),
            pl.BlockSpec((1, 32), full),
            pl.BlockSpec((32, 1), full),
            pl.BlockSpec((1, 1), full),
        ],
        out_specs=pl.BlockSpec((_BBLK, 1), row),
        out_shape=jax.ShapeDtypeStruct((B, 1), jnp.float32),
    )(u2, i2, su, si, w1u, w1i, b1, w2, b2, w3, b3, w4, b4)


def kernel(user_ids, item_ids, user_table, item_table,
           W1, b1, W2, b2, W3, b3, W4, b4):
    uids = user_ids.astype(jnp.int32)
    iids = item_ids.astype(jnp.int32)
    utab4 = _to_quads(user_table.T)
    itab4 = _to_quads(item_table.T)
    uquad = (uids // _TBLK) * _QBLK + (uids % _TBLK) % _QBLK
    iquad = (iids // _TBLK) * _QBLK + (iids % _TBLK) % _QBLK
    u2, i2 = _sc_gather(uquad, iquad, utab4, itab4)
    su = ((uids % _TBLK) // _QBLK).astype(jnp.int32)[:, None]
    si = ((iids % _TBLK) // _QBLK).astype(jnp.int32)[:, None]
    w1t = W1.T
    out = _mlp(u2, i2, su, si, w1t[:D], w1t[D:], b1[None, :], W2.T,
               b2[None, :], W3.T, b3[None, :], W4.T, b4[None, :])
    return out[:, 0]


# speedup vs baseline: 3.4101x; 1.0122x over previous
import functools

import jax
import jax.numpy as jnp
from jax import lax
from jax.experimental import pallas as pl
from jax.experimental.pallas import tpu as pltpu
from jax.experimental.pallas import tpu_sc as plsc

B = 16384
D = 64
NROWS = 1000000

_info = plsc.get_sparse_core_info()
_NC, _NS = _info.num_cores, _info.num_subcores
_NW = _NC * _NS
_BPW = B // _NW
_CHUNK = 128
_NCHUNK = _BPW // _CHUNK

_sc_mesh = plsc.VectorSubcoreMesh(core_axis_name="c", subcore_axis_name="s")


_TBLK = 49152
_QBLK = _TBLK // 4
_TGRID = pl.cdiv(NROWS, _TBLK)
NQUAD = _TGRID * _QBLK

_HI = 0xFFFF0000


def _bf16_hi_bits(x):
    y = x.astype(jnp.bfloat16).astype(jnp.float32)
    return jax.lax.bitcast_convert_type(y, jnp.uint32)


def _tr_body(xT_ref, o_ref):
    x = xT_ref[...]
    xs = jnp.concatenate(
        [x[:, i * _QBLK:(i + 1) * _QBLK] for i in range(4)], axis=0)
    t = xs.T
    a = _bf16_hi_bits(t[:, :D])
    b = _bf16_hi_bits(t[:, D:2 * D])
    c = _bf16_hi_bits(t[:, 2 * D:3 * D])
    d = _bf16_hi_bits(t[:, 3 * D:])
    wab = (a >> jnp.uint32(16)) | (b & jnp.uint32(_HI))
    wcd = (c >> jnp.uint32(16)) | (d & jnp.uint32(_HI))
    packed = jnp.concatenate([wab, wcd], axis=1)
    o_ref[...] = jax.lax.bitcast_convert_type(packed, jnp.float32)


def _to_quads(tabT):
    return pl.pallas_call(
        _tr_body,
        grid=(_TGRID,),
        in_specs=[pl.BlockSpec((D, _TBLK), lambda g: (0, g))],
        out_specs=pl.BlockSpec((_QBLK, 2 * D), lambda g: (g, 0)),
        out_shape=jax.ShapeDtypeStruct((NQUAD, 2 * D), jnp.float32),
    )(tabT)


@functools.partial(
    pl.kernel,
    mesh=_sc_mesh,
    out_type=[
        jax.ShapeDtypeStruct((B, 2 * D), jnp.float32),
        jax.ShapeDtypeStruct((B, 2 * D), jnp.float32),
    ],
    scratch_types=[
        pltpu.VMEM((_NCHUNK, _CHUNK), jnp.int32),
        pltpu.VMEM((_BPW, 2 * D), jnp.float32),
        pltpu.SemaphoreType.DMA,
    ],
)
def _sc_gather(uid_hbm, iid_hbm, utab_hbm, itab_hbm, u_out, i_out,
               idx_v, rows_v, sem):
    wid = lax.axis_index("s") * _NC + lax.axis_index("c")
    base = wid * _BPW
    for tab_hbm, ids_hbm, out_hbm in ((utab_hbm, uid_hbm, u_out),
                                      (itab_hbm, iid_hbm, i_out)):
        for j in range(_NCHUNK):
            pltpu.sync_copy(ids_hbm.at[pl.ds(base + j * _CHUNK, _CHUNK)],
                            idx_v.at[j])
        copies = [
            pltpu.async_copy(tab_hbm.at[idx_v.at[j]],
                             rows_v.at[pl.ds(j * _CHUNK, _CHUNK)], sem)
            for j in range(_NCHUNK)
        ]
        for c in copies:
            c.wait()
        pltpu.sync_copy(rows_v, out_hbm.at[pl.ds(base, _BPW)])


def _unpack(v2, wsel, lohi):
    v = jax.lax.bitcast_convert_type(v2, jnp.uint32)
    half = jnp.where(wsel, v[:, D:], v[:, :D])
    bits = jnp.where(lohi, half & jnp.uint32(_HI), half << jnp.uint32(16))
    return jax.lax.bitcast_convert_type(bits, jnp.float32)


def _mlp_body(u2_ref, i2_ref, su_ref, si_ref, w1u_ref, w1i_ref, b1_ref,
              w2_ref, b2_ref, w3_ref, b3_ref, w4_ref, b4_ref, o_ref):
    su = su_ref[...]
    si = si_ref[...]
    u = _unpack(u2_ref[...], su >= 2, (su % 2) == 1)
    i = _unpack(i2_ref[...], si >= 2, (si % 2) == 1)
    h = jnp.dot(u, w1u_ref[...], preferred_element_type=jnp.float32)
    h = h + jnp.dot(i, w1i_ref[...], preferred_element_type=jnp.float32)
    h = jnp.maximum(h + b1_ref[...], 0.0)
    h = jnp.maximum(
        jnp.dot(h, w2_ref[...], preferred_element_type=jnp.float32)
        + b2_ref[...], 0.0)
    h = jnp.maximum(
        jnp.dot(h, w3_ref[...], preferred_element_type=jnp.float32)
        + b3_ref[...], 0.0)
    z = (jnp.dot(h, w4_ref[...], preferred_element_type=jnp.float32)
         + b4_ref[...])
    o_ref[...] = jax.nn.sigmoid(z)


_BBLK = 2048


def _mlp(u2, i2, su, si, w1u, w1i, b1, w2, b2, w3, b3, w4, b4):
    grid = (B // _BBLK,)
    full = lambda gi: (0, 0)
    row = lambda gi: (gi, 0)
    return pl.pallas_call(
        _mlp_body,
        grid=grid,
        in_specs=[
            pl.BlockSpec((_BBLK, 2 * D), row),
            pl.BlockSpec((_BBLK, 2 * D), row),
            pl.BlockSpec((_BBLK, 1), row),
            pl.BlockSpec((_BBLK, 1), row),
            pl.BlockSpec((D, 128), full),
            pl.BlockSpec((D, 128), full),
            pl.BlockSpec((1, 128), full),
            pl.BlockSpec((128, 64), full),
            pl.BlockSpec((1, 64), full),
            pl.BlockSpec((64, 32), full),
            pl.BlockSpec((1, 32), full),
            pl.BlockSpec((32, 1), full),
            pl.BlockSpec((1, 1), full),
        ],
        out_specs=pl.BlockSpec((_BBLK, 1), row),
        out_shape=jax.ShapeDtypeStruct((B, 1), jnp.float32),
    )(u2, i2, su, si, w1u, w1i, b1, w2, b2, w3, b3, w4, b4)


def kernel(user_ids, item_ids, user_table, item_table,
           W1, b1, W2, b2, W3, b3, W4, b4):
    uids = user_ids.astype(jnp.int32)
    iids = item_ids.astype(jnp.int32)
    utab4 = _to_quads(user_table.T)
    itab4 = _to_quads(item_table.T)
    uquad = (uids // _TBLK) * _QBLK + (uids % _TBLK) % _QBLK
    iquad = (iids // _TBLK) * _QBLK + (iids % _TBLK) % _QBLK
    u2, i2 = _sc_gather(uquad, iquad, utab4, itab4)
    su = ((uids % _TBLK) // _QBLK).astype(jnp.int32)[:, None]
    si = ((iids % _TBLK) // _QBLK).astype(jnp.int32)[:, None]
    w1t = W1.T
    out = _mlp(u2, i2, su, si, w1t[:D], w1t[D:], b1[None, :], W2.T,
               b2[None, :], W3.T, b3[None, :], W4.T, b4[None, :])
    return out[:, 0]
